# baseline (speedup 1.0000x reference)
import functools

import jax
import jax.numpy as jnp
from jax import lax
from jax.experimental import pallas as pl
from jax.experimental.pallas import tpu as pltpu
from jax.experimental.pallas import tpu_sc as plsc

_B, _C, _N = 8, 190, 16384
_NC, _NS, _L = 2, 16, 16
_NW = _NC * _NS

_N_SC = 4096
_W = _N_SC // _NW
_NG = _W // _L
_NSTRIPE = 4
_CS = _C // _NSTRIPE
_TOTAL = _B

_NB1 = 2048
_N_TC1 = 10240
_NB2 = 2048
_N_TC2 = _N - _N_SC - _N_TC1

_mesh = plsc.VectorSubcoreMesh(core_axis_name="c", subcore_axis_name="s")


def _merge(a, b):
    a1, a2 = a
    b1, b2 = b
    hi = jnp.maximum(a1, b1)
    lo = jnp.maximum(jnp.minimum(a1, b1), jnp.maximum(a2, b2))
    return hi, lo


@functools.partial(
    pl.kernel,
    mesh=_mesh,
    out_type=jax.ShapeDtypeStruct((_NW, _B, _L), jnp.float32),
    scratch_types=[
        pltpu.VMEM((2, _C, _W), jnp.float32),
        pltpu.VMEM((_B, _L), jnp.float32),
        pltpu.SemaphoreType.DMA,
    ],
)
def _sc_conf(sim_hbm, out_hbm, bufall, acc_v, sem):
    wid = lax.axis_index("s") * _NC + lax.axis_index("c")
    tok0 = wid * _W
    neg = jnp.full((_L,), -jnp.inf, jnp.float32)

    def chunk_copy(b):
        par = lax.rem(b, 2)
        return pltpu.make_async_copy(
            sim_hbm.at[b, :, pl.ds(tok0, _W)], bufall.at[par], sem
        )

    chunk_copy(0).start()
    for b0 in range(_B):
        acc_v[b0, :] = jnp.zeros((_L,), jnp.float32)

    def chunk_body(b, _):
        chunk_copy(b).wait()

        @pl.when(b + 1 < _TOTAL)
        def _start_next():
            chunk_copy(b + 1).start()

        par = lax.rem(b, 2)

        def group_body(g, acc):
            sl = pl.ds(g * _L, _L)

            def chan_body(c, carry):
                new = []
                for s in range(_NSTRIPE):
                    v = bufall[par, c * _NSTRIPE + s, sl]
                    m1, m2 = carry[s]
                    m2 = jnp.maximum(m2, jnp.minimum(m1, v))
                    m1 = jnp.maximum(m1, v)
                    new.append((m1, m2))
                return tuple(new)

            init = tuple((neg, neg) for _ in range(_NSTRIPE))
            stripes = lax.fori_loop(0, _CS, chan_body, init)
            m1, m2 = stripes[0]
            for s in range(1, _NSTRIPE):
                m1, m2 = _merge((m1, m2), stripes[s])
            for c in range(_CS * _NSTRIPE, _C):
                v = bufall[par, c, sl]
                m2 = jnp.maximum(m2, jnp.minimum(m1, v))
                m1 = jnp.maximum(m1, v)
            conf = jnp.exp(1.0 - m1 / (m2 + 1e-8))
            return acc + conf

        acc = lax.fori_loop(0, _NG, group_body, jnp.zeros((_L,), jnp.float32))
        acc_v[b, :] = acc_v[b, :] + acc
        return 0

    lax.fori_loop(0, _TOTAL, chunk_body, 0)
    pltpu.sync_copy(acc_v, out_hbm.at[wid])


def _tc_body(x_ref, out_ref):
    x = x_ref[0]
    m1 = jnp.max(x, axis=0)
    is_max = x == m1[None, :]
    cnt = jnp.sum(is_max.astype(jnp.float32), axis=0)
    neg = jnp.float32(-jnp.inf)
    m2c = jnp.max(jnp.where(is_max, neg, x), axis=0)
    m2 = jnp.where(cnt > 1.0, m1, m2c)
    conf = jnp.exp(1.0 - m1 / (m2 + 1e-8))
    out_ref[0, 0, :] = conf


def _tc_call(sim_mat, tok_start, n_tok, nb):
    nblk = n_tok // nb
    blk0 = tok_start // nb
    return pl.pallas_call(
        _tc_body,
        grid=(_B, nblk),
        in_specs=[pl.BlockSpec((1, _C, nb), lambda b, n: (b, 0, n + blk0))],
        out_specs=pl.BlockSpec((1, 1, nb), lambda b, n: (b * nblk + n, 0, 0)),
        out_shape=jax.ShapeDtypeStruct((_B * nblk, 1, nb), jnp.float32),
    )(sim_mat).reshape(_B, nblk * nb)


def kernel(sim_mat):
    sc_out = _sc_conf(sim_mat)
    tc1 = _tc_call(sim_mat, _N_SC, _N_TC1, _NB1)
    tc2 = _tc_call(sim_mat, _N_SC + _N_TC1, _N_TC2, _NB2)
    total = sc_out.sum(axis=(0, 2)) + tc1.sum(axis=-1) + tc2.sum(axis=-1)
    return total / _N


# speedup vs baseline: 1.2181x; 1.2181x over previous
import jax
import jax.numpy as jnp
from jax.experimental import pallas as pl

_B, _C, _N = 8, 190, 16384
_NB = 8192


def _conf_body(x_ref, out_ref):
    x = x_ref[0]
    m1 = jnp.max(x, axis=0)
    is_max = x == m1[None, :]
    cnt = jnp.sum(is_max.astype(jnp.float32), axis=0)
    neg = jnp.float32(-jnp.inf)
    m2c = jnp.max(jnp.where(is_max, neg, x), axis=0)
    m2 = jnp.where(cnt > 1.0, m1, m2c)
    conf = jnp.exp(1.0 - m1 / (m2 + 1e-8))
    out_ref[0, 0, :] = conf


def kernel(sim_mat):
    nblk = _N // _NB
    conf = pl.pallas_call(
        _conf_body,
        grid=(_B, nblk),
        in_specs=[pl.BlockSpec((1, _C, _NB), lambda b, n: (b, 0, n))],
        out_specs=pl.BlockSpec((1, 1, _NB), lambda b, n: (b * nblk + n, 0, 0)),
        out_shape=jax.ShapeDtypeStruct((_B * nblk, 1, _NB), jnp.float32),
    )(sim_mat)
    return jnp.mean(conf.reshape(_B, nblk * _NB), axis=-1)


# speedup vs baseline: 3.7280x; 3.0606x over previous
import jax
import jax.numpy as jnp
from jax.experimental import pallas as pl

_B, _C, _N = 8, 190, 16384
_NB = 1024


def _conf_body(x_ref, out_ref):
    x = x_ref[...]
    m1 = jnp.max(x, axis=0)
    is_max = x == m1[None]
    cnt = jnp.sum(is_max.astype(jnp.float32), axis=0)
    neg = jnp.float32(-jnp.inf)
    m2c = jnp.max(jnp.where(is_max, neg, x), axis=0)
    m2 = jnp.where(cnt > 1.0, m1, m2c)
    conf = jnp.exp(1.0 - m1 / (m2 + 1e-8))
    out_ref[0] = conf


def kernel(sim_mat):
    xt = jnp.transpose(sim_mat, (1, 0, 2))
    nblk = _N // _NB
    conf = pl.pallas_call(
        _conf_body,
        grid=(nblk,),
        in_specs=[pl.BlockSpec((_C, _B, _NB), lambda n: (0, 0, n))],
        out_specs=pl.BlockSpec((1, _B, _NB), lambda n: (n, 0, 0)),
        out_shape=jax.ShapeDtypeStruct((nblk, _B, _NB), jnp.float32),
    )(xt)
    return jnp.mean(conf, axis=(0, 2))


# speedup vs baseline: 4.0374x; 1.0830x over previous
import jax
import jax.numpy as jnp
from jax.experimental import pallas as pl

_B, _C, _N = 8, 190, 16384
_NB = 2048


def _conf_body(x_ref, out_ref):
    x = x_ref[...]
    m1 = jnp.max(x, axis=0)
    is_max = x == m1[None]
    cnt = jnp.sum(is_max.astype(jnp.float32), axis=0)
    neg = jnp.float32(-jnp.inf)
    m2c = jnp.max(jnp.where(is_max, neg, x), axis=0)
    m2 = jnp.where(cnt > 1.0, m1, m2c)
    conf = jnp.exp(1.0 - m1 / (m2 + 1e-8))
    out_ref[0] = conf


def kernel(sim_mat):
    xt = jnp.transpose(sim_mat, (1, 0, 2))
    nblk = _N // _NB
    conf = pl.pallas_call(
        _conf_body,
        grid=(nblk,),
        in_specs=[pl.BlockSpec((_C, _B, _NB), lambda n: (0, 0, n))],
        out_specs=pl.BlockSpec((1, _B, _NB), lambda n: (n, 0, 0)),
        out_shape=jax.ShapeDtypeStruct((nblk, _B, _NB), jnp.float32),
    )(xt)
    return jnp.mean(conf, axis=(0, 2))
